# baseline (speedup 1.0000x reference)
import jax
import jax.numpy as jnp
from jax.experimental import pallas as pl

_PADDING_IDX = 0
_S_BLK = 2048
_B_BLK = 2


def _body(in_ref, w_ref, out_ref):
    w = w_ref[...]
    for b in range(_B_BLK):
        mask = in_ref[0, :, b:b + 1] != _PADDING_IDX
        out_ref[b] = jnp.where(mask, w, 0.0)


def kernel(input, weights):
    bsz, seq_len = input.shape
    dim = weights.shape[1]
    num_s = seq_len // _S_BLK
    inp_t = input.T.reshape(seq_len, bsz // _B_BLK, _B_BLK).transpose(1, 0, 2)
    return pl.pallas_call(
        _body,
        grid=(num_s, bsz // _B_BLK),
        in_specs=[
            pl.BlockSpec((1, _S_BLK, _B_BLK), lambda s, p: (p, s, 0)),
            pl.BlockSpec((_S_BLK, dim), lambda s, p: (s, 0)),
        ],
        out_specs=pl.BlockSpec((_B_BLK, _S_BLK, dim), lambda s, p: (p, s, 0)),
        out_shape=jax.ShapeDtypeStruct((bsz, seq_len, dim), weights.dtype),
    )(inp_t, weights)


# speedup vs baseline: 1.0802x; 1.0802x over previous
import jax
import jax.numpy as jnp
from jax.experimental import pallas as pl

_PADDING_IDX = 0
_S_BLK = 1024


def _body(in_ref, w_ref, out_ref):
    w = w_ref[...]
    for b in range(out_ref.shape[0]):
        mask = in_ref[:, b:b + 1] != _PADDING_IDX
        out_ref[b] = jnp.where(mask, w, 0.0)


def kernel(input, weights):
    bsz, seq_len = input.shape
    dim = weights.shape[1]
    num_s = seq_len // _S_BLK
    inp_t = input.T
    return pl.pallas_call(
        _body,
        grid=(num_s,),
        in_specs=[
            pl.BlockSpec((_S_BLK, bsz), lambda s: (s, 0)),
            pl.BlockSpec((_S_BLK, dim), lambda s: (s, 0)),
        ],
        out_specs=pl.BlockSpec((bsz, _S_BLK, dim), lambda s: (0, s, 0)),
        out_shape=jax.ShapeDtypeStruct((bsz, seq_len, dim), weights.dtype),
    )(inp_t, weights)
